# Initial kernel scaffold; baseline (speedup 1.0000x reference)
import jax
import jax.numpy as jnp
from jax.experimental import pallas as pl


def kernel(logits, targets):
    raise NotImplementedError("write your pallas kernel here")


# speedup vs baseline: 19.2612x; 19.2612x over previous
import functools

import jax
import jax.numpy as jnp
from jax import lax
from jax.experimental import pallas as pl
from jax.experimental.pallas import tpu as pltpu
from jax.experimental.pallas import tpu_sc as plsc

SAMPLES = 16
ELEMS = 512 * 512
HALF = ELEMS // 2
CHUNK = 16384
NCHUNK = HALF // CHUNK
ITERS = CHUNK // 16
MBITS = 5
NEXP = 32
NBINS = NEXP << MBITS
SHIFT = 23 - MBITS
OFFSET = 112 << MBITS
HSIZE = 3 * NBINS * 16
HSMALL = 3 * NBINS


def _sc_body(logf, tgtf, zeros, hist_out, p_out, buf_l, buf_t, hist, hsmall,
             pvec):
    s = lax.axis_index("s")
    c = lax.axis_index("c")
    wid = s * 2 + c
    base = wid * HALF
    pltpu.sync_copy(zeros, hist)
    lane = lax.iota(jnp.int32, 16)

    def chunk_body(ci, carry):
        off = base + ci * CHUNK
        pltpu.sync_copy(logf.at[pl.ds(off, CHUNK)], buf_l)
        pltpu.sync_copy(tgtf.at[pl.ds(off, CHUNK)], buf_t)

        def inner(i, carry2):
            lg = buf_l[pl.ds(i * 16, 16)]
            tg = buf_t[pl.ds(i * 16, 16)]
            tf = tg.astype(jnp.float32)
            e = 1.0 - lg * (2.0 * tf - 1.0)
            x = jnp.maximum(e, 0.0)
            b = (lax.bitcast_convert_type(x, jnp.int32) >> SHIFT) - OFFSET
            b = jnp.clip(b, 0, NBINS - 1)
            addr = b * 16 + lane
            one = (x > 0.0).astype(jnp.float32)
            negv = one * (1.0 - tf)
            plsc.addupdate_scatter(hist, [addr], one)
            plsc.addupdate_scatter(hist, [addr + (NBINS * 16)], negv)
            plsc.addupdate_scatter(hist, [addr + (2 * NBINS * 16)], x)
            return carry2 + tf

        return lax.fori_loop(0, ITERS, inner, carry)

    pacc = lax.fori_loop(0, NCHUNK, chunk_body, jnp.zeros((16,), jnp.float32))
    pvec[...] = pacc

    def red_body(g, carry):
        pb = g * 16 + lane

        def lred(l, acc):
            return acc + plsc.load_gather(hist, [pb * 16 + l])

        acc = lax.fori_loop(0, 16, lred, jnp.zeros((16,), jnp.float32))
        hsmall[pl.ds(g * 16, 16)] = acc
        return carry

    lax.fori_loop(0, HSMALL // 16, red_body, jnp.int32(0))

    pltpu.sync_copy(hsmall, hist_out.at[pl.ds(wid * HSMALL, HSMALL)])
    pltpu.sync_copy(pvec, p_out.at[pl.ds(wid * 16, 16)])


@functools.cache
def _get_sc_hist():
    return functools.partial(
        pl.kernel,
        mesh=plsc.VectorSubcoreMesh(core_axis_name="c", subcore_axis_name="s"),
        compiler_params=pltpu.CompilerParams(needs_layout_passes=False),
        out_type=(
            jax.ShapeDtypeStruct((32 * HSMALL,), jnp.float32),
            jax.ShapeDtypeStruct((32 * 16,), jnp.float32),
        ),
        scratch_types=[
            pltpu.VMEM((CHUNK,), jnp.float32),
            pltpu.VMEM((CHUNK,), jnp.int32),
            pltpu.VMEM((HSIZE,), jnp.float32),
            pltpu.VMEM((HSMALL,), jnp.float32),
            pltpu.VMEM((16,), jnp.float32),
        ],
    )(_sc_body)


def _tc_body(h_ref, p_ref, o_ref):
    h = h_ref[:]
    h2 = h[:, 0] + h[:, 1]
    cnt = h2[:, 0]
    neg = h2[:, 1]
    sumx = h2[:, 2]
    p = p_ref[:]
    ptot = jnp.sum(jnp.sum(p, axis=2), axis=1, keepdims=True)

    def cum(a):
        sft = 1
        while sft < NBINS:
            a = a + jnp.concatenate(
                [jnp.zeros((SAMPLES, sft), jnp.float32), a[:, :-sft]], axis=1)
            sft *= 2
        return a

    ic = cum(cnt)
    inm = cum(neg)
    n_gt = ic[:, NBINS - 1:] - ic
    m_gt = inm[:, NBINS - 1:] - inm
    k = lax.broadcasted_iota(jnp.int32, (SAMPLES, NBINS), 1)
    a_lo = lax.bitcast_convert_type((k + OFFSET) << SHIFT, jnp.float32)
    a_hi = lax.bitcast_convert_type((k + 1 + OFFSET) << SHIFT, jnp.float32)
    hw = a_hi - a_lo
    frac = jnp.maximum(sumx - a_lo * cnt, 0.0)
    num = hw * n_gt + frac
    den = ptot + m_gt + 0.5 * neg
    contrib = jnp.where(den > 0.0, num / den, 0.0)
    o_ref[...] = jnp.sum(contrib, keepdims=True) / SAMPLES


def kernel(logits, targets):
    logf = logits.reshape(-1)
    tgtf = targets.reshape(-1)
    zeros = jnp.zeros((HSIZE,), jnp.float32)
    hist_out, p_out = _get_sc_hist()(logf, tgtf, zeros)
    h4 = hist_out.reshape(SAMPLES, 2, 3, NBINS)
    p3 = p_out.reshape(SAMPLES, 2, 16)
    out = pl.pallas_call(
        _tc_body,
        out_shape=jax.ShapeDtypeStruct((1, 1), jnp.float32),
    )(h4, p3)
    return out[0, 0]


# speedup vs baseline: 22.1715x; 1.1511x over previous
import functools

import jax
import jax.numpy as jnp
from jax import lax
from jax.experimental import pallas as pl
from jax.experimental.pallas import tpu as pltpu
from jax.experimental.pallas import tpu_sc as plsc

SAMPLES = 16
ELEMS = 512 * 512
HALF = ELEMS // 2
CHUNK = 16384
NCHUNK = HALF // CHUNK
ITERS = CHUNK // 16
MBITS = 5
NEXP = 32
NBINS = NEXP << MBITS
SHIFT = 23 - MBITS
OFFSET = 112 << MBITS
HSIZE = 3 * NBINS * 16
HSMALL = 3 * NBINS


UNROLL = 8


def _sc_body(logf, tgtf, hist_out, p_out, buf_l0, buf_t0, buf_l1, buf_t1,
             hist, hsmall, pvec, sl0, st0, sl1, st1):
    s = lax.axis_index("s")
    c = lax.axis_index("c")
    wid = s * 2 + c
    base = wid * HALF
    lane = lax.iota(jnp.int32, 16)
    zero16 = jnp.zeros((16,), jnp.float32)

    def z_body(i, carry):
        for u in range(16):
            hist[pl.ds(i * 256 + u * 16, 16)] = zero16
        return carry

    lax.fori_loop(0, HSIZE // 256, z_body, jnp.int32(0))

    bufs = ((buf_l0, buf_t0, sl0, st0), (buf_l1, buf_t1, sl1, st1))

    def start(ci, parity):
        bl, bt, sl, st = bufs[parity]
        off = base + ci * CHUNK
        hl = pltpu.async_copy(logf.at[pl.ds(off, CHUNK)], bl, sl)
        ht = pltpu.async_copy(tgtf.at[pl.ds(off, CHUNK)], bt, st)
        return hl, ht

    handles = [None, None]
    handles[0] = start(0, 0)
    pacc = zero16
    for ci in range(NCHUNK):
        par = ci % 2
        if ci + 1 < NCHUNK:
            handles[1 - par] = start(ci + 1, 1 - par)
        hl, ht = handles[par]
        hl.wait()
        ht.wait()
        bl, bt = bufs[par][0], bufs[par][1]

        def inner(j, pc, bl=bl, bt=bt):
            psum = zero16
            for u in range(UNROLL):
                o = j * (UNROLL * 16) + u * 16
                lg = bl[pl.ds(o, 16)]
                tg = bt[pl.ds(o, 16)]
                tf = tg.astype(jnp.float32)
                e = 1.0 - lg * (2.0 * tf - 1.0)
                x = jnp.maximum(e, 0.0)
                b = (lax.bitcast_convert_type(x, jnp.int32) >> SHIFT) - OFFSET
                b = jnp.clip(b, 0, NBINS - 1)
                addr = b * 16 + lane
                one = (x > 0.0).astype(jnp.float32)
                negv = one * (1.0 - tf)
                plsc.addupdate_scatter(hist, [addr], one)
                plsc.addupdate_scatter(hist, [addr + (NBINS * 16)], negv)
                plsc.addupdate_scatter(hist, [addr + (2 * NBINS * 16)], x)
                psum = psum + tf
            return pc + psum

        pacc = lax.fori_loop(0, ITERS // UNROLL, inner, pacc)

    pvec[...] = pacc

    def red_body(g, carry):
        pb16 = (g * 16 + lane) * 16
        acc = zero16
        for l in range(16):
            acc = acc + plsc.load_gather(hist, [pb16 + l])
        hsmall[pl.ds(g * 16, 16)] = acc
        return carry

    lax.fori_loop(0, HSMALL // 16, red_body, jnp.int32(0))

    pltpu.sync_copy(hsmall, hist_out.at[pl.ds(wid * HSMALL, HSMALL)])
    pltpu.sync_copy(pvec, p_out.at[pl.ds(wid * 16, 16)])


@functools.cache
def _get_sc_hist():
    return functools.partial(
        pl.kernel,
        mesh=plsc.VectorSubcoreMesh(core_axis_name="c", subcore_axis_name="s"),
        compiler_params=pltpu.CompilerParams(needs_layout_passes=False),
        out_type=(
            jax.ShapeDtypeStruct((32 * HSMALL,), jnp.float32),
            jax.ShapeDtypeStruct((32 * 16,), jnp.float32),
        ),
        scratch_types=[
            pltpu.VMEM((CHUNK,), jnp.float32),
            pltpu.VMEM((CHUNK,), jnp.int32),
            pltpu.VMEM((CHUNK,), jnp.float32),
            pltpu.VMEM((CHUNK,), jnp.int32),
            pltpu.VMEM((HSIZE,), jnp.float32),
            pltpu.VMEM((HSMALL,), jnp.float32),
            pltpu.VMEM((16,), jnp.float32),
            pltpu.SemaphoreType.DMA,
            pltpu.SemaphoreType.DMA,
            pltpu.SemaphoreType.DMA,
            pltpu.SemaphoreType.DMA,
        ],
    )(_sc_body)


def _tc_body(h_ref, p_ref, o_ref):
    h = h_ref[:]
    h2 = h[:, 0] + h[:, 1]
    cnt = h2[:, 0]
    neg = h2[:, 1]
    sumx = h2[:, 2]
    p = p_ref[:]
    ptot = jnp.sum(jnp.sum(p, axis=2), axis=1, keepdims=True)

    def cum(a):
        sft = 1
        while sft < NBINS:
            a = a + jnp.concatenate(
                [jnp.zeros((SAMPLES, sft), jnp.float32), a[:, :-sft]], axis=1)
            sft *= 2
        return a

    ic = cum(cnt)
    inm = cum(neg)
    n_gt = ic[:, NBINS - 1:] - ic
    m_gt = inm[:, NBINS - 1:] - inm
    k = lax.broadcasted_iota(jnp.int32, (SAMPLES, NBINS), 1)
    a_lo = lax.bitcast_convert_type((k + OFFSET) << SHIFT, jnp.float32)
    a_hi = lax.bitcast_convert_type((k + 1 + OFFSET) << SHIFT, jnp.float32)
    hw = a_hi - a_lo
    frac = jnp.maximum(sumx - a_lo * cnt, 0.0)
    num = hw * n_gt + frac
    den = ptot + m_gt + 0.5 * neg
    contrib = jnp.where(den > 0.0, num / den, 0.0)
    o_ref[...] = jnp.sum(contrib, keepdims=True) / SAMPLES


def kernel(logits, targets):
    logf = logits.reshape(-1)
    tgtf = targets.reshape(-1)
    hist_out, p_out = _get_sc_hist()(logf, tgtf)
    h4 = hist_out.reshape(SAMPLES, 2, 3, NBINS)
    p3 = p_out.reshape(SAMPLES, 2, 16)
    out = pl.pallas_call(
        _tc_body,
        out_shape=jax.ShapeDtypeStruct((1, 1), jnp.float32),
    )(h4, p3)
    return out[0, 0]


# speedup vs baseline: 41.9653x; 1.8928x over previous
import functools

import jax
import jax.numpy as jnp
from jax import lax
from jax.experimental import pallas as pl
from jax.experimental.pallas import tpu as pltpu
from jax.experimental.pallas import tpu_sc as plsc

SAMPLES = 16
ELEMS = 512 * 512
HALF = ELEMS // 2
CHUNK = 16384
NCHUNK = HALF // CHUNK
ITERS = CHUNK // 16
MBITS = 5
NEXP = 16
NBINS = NEXP << MBITS
SHIFT = 23 - MBITS
OFFSET = 119 << MBITS
HCOPY = 3 * NBINS * 16
HSIZE = 2 * HCOPY
HSMALL = 3 * NBINS


UNROLL = 8


def _sc_body(logf, tgtf, hist_out, p_out, buf_l0, buf_t0, buf_l1, buf_t1,
             hist, hsmall, pvec, sl0, st0, sl1, st1):
    s = lax.axis_index("s")
    c = lax.axis_index("c")
    wid = s * 2 + c
    base = wid * HALF
    lane = lax.iota(jnp.int32, 16)
    zero16 = jnp.zeros((16,), jnp.float32)

    def z_body(i, carry):
        for u in range(16):
            hist[pl.ds(i * 256 + u * 16, 16)] = zero16
        return carry

    lax.fori_loop(0, HSIZE // 256, z_body, jnp.int32(0))

    bufs = ((buf_l0, buf_t0, sl0, st0), (buf_l1, buf_t1, sl1, st1))

    def start(ci, parity):
        bl, bt, sl, st = bufs[parity]
        off = base + ci * CHUNK
        hl = pltpu.async_copy(logf.at[pl.ds(off, CHUNK)], bl, sl)
        ht = pltpu.async_copy(tgtf.at[pl.ds(off, CHUNK)], bt, st)
        return hl, ht

    handles = [None, None]
    handles[0] = start(0, 0)
    pacc = zero16
    for ci in range(NCHUNK):
        par = ci % 2
        if ci + 1 < NCHUNK:
            handles[1 - par] = start(ci + 1, 1 - par)
        hl, ht = handles[par]
        hl.wait()
        ht.wait()
        bl, bt = bufs[par][0], bufs[par][1]

        def inner(j, pc, bl=bl, bt=bt):
            psum = zero16
            vals = []
            for u in range(UNROLL):
                o = j * (UNROLL * 16) + u * 16
                lg = bl[pl.ds(o, 16)]
                tg = bt[pl.ds(o, 16)]
                tf = tg.astype(jnp.float32)
                e = 1.0 - lg * (2.0 * tf - 1.0)
                x = jnp.maximum(e, 0.0)
                b = (lax.bitcast_convert_type(x, jnp.int32) >> SHIFT) - OFFSET
                b = jnp.clip(b, 0, NBINS - 1)
                addr = b * 16 + lane + (u % 2) * HCOPY
                one = (x > 0.0).astype(jnp.float32)
                negv = one * (1.0 - tf)
                vals.append((addr, one, negv, x))
                psum = psum + tf
            for addr, one, negv, x in vals:
                plsc.addupdate_scatter(hist, [addr], one)
                plsc.addupdate_scatter(hist, [addr + (NBINS * 16)], negv)
                plsc.addupdate_scatter(hist, [addr + (2 * NBINS * 16)], x)
            return pc + psum

        pacc = lax.fori_loop(0, ITERS // UNROLL, inner, pacc)

    pvec[...] = pacc

    def red_body(g, carry):
        pb16 = (g * 16 + lane) * 16
        acc = zero16
        for cp in (0, HCOPY):
            for l in range(16):
                acc = acc + plsc.load_gather(hist, [pb16 + cp + l])
        hsmall[pl.ds(g * 16, 16)] = acc
        return carry

    lax.fori_loop(0, HSMALL // 16, red_body, jnp.int32(0))

    pltpu.sync_copy(hsmall, hist_out.at[pl.ds(wid * HSMALL, HSMALL)])
    pltpu.sync_copy(pvec, p_out.at[pl.ds(wid * 16, 16)])


@functools.cache
def _get_sc_hist():
    return functools.partial(
        pl.kernel,
        mesh=plsc.VectorSubcoreMesh(core_axis_name="c", subcore_axis_name="s"),
        compiler_params=pltpu.CompilerParams(needs_layout_passes=False),
        out_type=(
            jax.ShapeDtypeStruct((32 * HSMALL,), jnp.float32),
            jax.ShapeDtypeStruct((32 * 16,), jnp.float32),
        ),
        scratch_types=[
            pltpu.VMEM((CHUNK,), jnp.float32),
            pltpu.VMEM((CHUNK,), jnp.int32),
            pltpu.VMEM((CHUNK,), jnp.float32),
            pltpu.VMEM((CHUNK,), jnp.int32),
            pltpu.VMEM((HSIZE,), jnp.float32),
            pltpu.VMEM((HSMALL,), jnp.float32),
            pltpu.VMEM((16,), jnp.float32),
            pltpu.SemaphoreType.DMA,
            pltpu.SemaphoreType.DMA,
            pltpu.SemaphoreType.DMA,
            pltpu.SemaphoreType.DMA,
        ],
    )(_sc_body)


def _tc_body(h_ref, p_ref, o_ref):
    h = h_ref[:]
    h2 = h[:, 0] + h[:, 1]
    cnt = h2[:, 0]
    neg = h2[:, 1]
    sumx = h2[:, 2]
    p = p_ref[:]
    ptot = jnp.sum(jnp.sum(p, axis=2), axis=1, keepdims=True)

    def cum(a):
        sft = 1
        while sft < NBINS:
            a = a + jnp.concatenate(
                [jnp.zeros((SAMPLES, sft), jnp.float32), a[:, :-sft]], axis=1)
            sft *= 2
        return a

    ic = cum(cnt)
    inm = cum(neg)
    n_gt = ic[:, NBINS - 1:] - ic
    m_gt = inm[:, NBINS - 1:] - inm
    k = lax.broadcasted_iota(jnp.int32, (SAMPLES, NBINS), 1)
    a_lo = lax.bitcast_convert_type((k + OFFSET) << SHIFT, jnp.float32)
    a_lo = jnp.where(k == 0, 0.0, a_lo)
    a_hi = lax.bitcast_convert_type((k + 1 + OFFSET) << SHIFT, jnp.float32)
    hw = a_hi - a_lo
    frac = jnp.maximum(sumx - a_lo * cnt, 0.0)
    num = hw * n_gt + frac
    den = ptot + m_gt + 0.5 * neg
    contrib = jnp.where(den > 0.0, num / den, 0.0)
    o_ref[...] = jnp.sum(contrib, keepdims=True) / SAMPLES


def kernel(logits, targets):
    logf = logits.reshape(-1)
    tgtf = targets.reshape(-1)
    hist_out, p_out = _get_sc_hist()(logf, tgtf)
    h4 = hist_out.reshape(SAMPLES, 2, 3, NBINS)
    p3 = p_out.reshape(SAMPLES, 2, 16)
    out = pl.pallas_call(
        _tc_body,
        out_shape=jax.ShapeDtypeStruct((1, 1), jnp.float32),
    )(h4, p3)
    return out[0, 0]


# speedup vs baseline: 58.8961x; 1.4034x over previous
import functools

import jax
import jax.numpy as jnp
from jax import lax
from jax.experimental import pallas as pl
from jax.experimental.pallas import tpu as pltpu
from jax.experimental.pallas import tpu_sc as plsc

SAMPLES = 16
ELEMS = 512 * 512
HALF = ELEMS // 2
CHUNK = 16384
NCHUNK = HALF // CHUNK
ITERS = CHUNK // 16
MBITS = 5
NEXP = 16
NBINS = NEXP << MBITS
SHIFT = 23 - MBITS
OFFSET = 119 << MBITS
HCOPY = 3 * NBINS * 16
HSIZE = 2 * HCOPY
HSMALL = 3 * NBINS


UNROLL = 8


def _sc_body(logf, tgtf, hist_out, p_out, buf_l0, buf_t0, buf_l1, buf_t1,
             hist, hsmall, pvec, sl0, st0, sl1, st1):
    s = lax.axis_index("s")
    c = lax.axis_index("c")
    wid = s * 2 + c
    lane = lax.iota(jnp.int32, 16)
    zero16 = jnp.zeros((16,), jnp.float32)

    def z_body(i, carry):
        for u in range(16):
            hist[pl.ds(i * 256 + u * 16, 16)] = zero16
        return carry

    lax.fori_loop(0, HSIZE // 256, z_body, jnp.int32(0))

    bufs = ((buf_l0, buf_t0, sl0, st0), (buf_l1, buf_t1, sl1, st1))
    ROWS = CHUNK // 512

    def start(ci, parity):
        bl, bt, sl, st = bufs[parity]
        row = c * (HALF // 512) + ci * ROWS
        hl = pltpu.async_copy(logf.at[s, pl.ds(row, ROWS), :], bl, sl)
        ht = pltpu.async_copy(tgtf.at[s, pl.ds(row, ROWS), :], bt, st)
        return hl, ht

    handles = [None, None]
    handles[0] = start(0, 0)
    pacc = zero16
    for ci in range(NCHUNK):
        par = ci % 2
        if ci + 1 < NCHUNK:
            handles[1 - par] = start(ci + 1, 1 - par)
        hl, ht = handles[par]
        hl.wait()
        ht.wait()
        bl, bt = bufs[par][0], bufs[par][1]

        def inner(j, pc, bl=bl, bt=bt):
            psum = zero16
            vals = []
            r = j >> 2
            cg = (j & 3) * 128
            for u in range(UNROLL):
                o = cg + u * 16
                lg = bl[r, pl.ds(o, 16)]
                tg = bt[r, pl.ds(o, 16)]
                tf = tg.astype(jnp.float32)
                e = 1.0 - lg * (2.0 * tf - 1.0)
                x = jnp.maximum(e, 0.0)
                b = (lax.bitcast_convert_type(x, jnp.int32) >> SHIFT) - OFFSET
                b = jnp.clip(b, 0, NBINS - 1)
                addr = b * 16 + lane + (u % 2) * HCOPY
                one = (x > 0.0).astype(jnp.float32)
                negv = one * (1.0 - tf)
                vals.append((addr, one, negv, x))
                psum = psum + tf
            for addr, one, negv, x in vals:
                plsc.addupdate_scatter(hist, [addr], one)
                plsc.addupdate_scatter(hist, [addr + (NBINS * 16)], negv)
                plsc.addupdate_scatter(hist, [addr + (2 * NBINS * 16)], x)
            return pc + psum

        pacc = lax.fori_loop(0, ITERS // UNROLL, inner, pacc)

    pvec[...] = pacc

    def red_body(g, carry):
        pb16 = (g * 16 + lane) * 16
        acc = zero16
        for cp in (0, HCOPY):
            for l in range(16):
                acc = acc + plsc.load_gather(hist, [pb16 + cp + l])
        hsmall[pl.ds(g * 16, 16)] = acc
        return carry

    lax.fori_loop(0, HSMALL // 16, red_body, jnp.int32(0))

    pltpu.sync_copy(hsmall, hist_out.at[pl.ds(wid * HSMALL, HSMALL)])
    pltpu.sync_copy(pvec, p_out.at[pl.ds(wid * 16, 16)])


@functools.cache
def _get_sc_hist():
    return functools.partial(
        pl.kernel,
        mesh=plsc.VectorSubcoreMesh(core_axis_name="c", subcore_axis_name="s"),
        compiler_params=pltpu.CompilerParams(needs_layout_passes=False),
        out_type=(
            jax.ShapeDtypeStruct((32 * HSMALL,), jnp.float32),
            jax.ShapeDtypeStruct((32 * 16,), jnp.float32),
        ),
        scratch_types=[
            pltpu.VMEM((CHUNK // 512, 512), jnp.float32),
            pltpu.VMEM((CHUNK // 512, 512), jnp.int32),
            pltpu.VMEM((CHUNK // 512, 512), jnp.float32),
            pltpu.VMEM((CHUNK // 512, 512), jnp.int32),
            pltpu.VMEM((HSIZE,), jnp.float32),
            pltpu.VMEM((HSMALL,), jnp.float32),
            pltpu.VMEM((16,), jnp.float32),
            pltpu.SemaphoreType.DMA,
            pltpu.SemaphoreType.DMA,
            pltpu.SemaphoreType.DMA,
            pltpu.SemaphoreType.DMA,
        ],
    )(_sc_body)


def _tc_body(h_ref, p_ref, o_ref):
    h = h_ref[:]
    h2 = h[:, 0] + h[:, 1]
    cnt = h2[:, 0]
    neg = h2[:, 1]
    sumx = h2[:, 2]
    p = p_ref[:]
    ptot = jnp.sum(jnp.sum(p, axis=2), axis=1, keepdims=True)

    def cum(a):
        sft = 1
        while sft < NBINS:
            a = a + jnp.concatenate(
                [jnp.zeros((SAMPLES, sft), jnp.float32), a[:, :-sft]], axis=1)
            sft *= 2
        return a

    ic = cum(cnt)
    inm = cum(neg)
    n_gt = ic[:, NBINS - 1:] - ic
    m_gt = inm[:, NBINS - 1:] - inm
    k = lax.broadcasted_iota(jnp.int32, (SAMPLES, NBINS), 1)
    a_lo = lax.bitcast_convert_type((k + OFFSET) << SHIFT, jnp.float32)
    a_lo = jnp.where(k == 0, 0.0, a_lo)
    a_hi = lax.bitcast_convert_type((k + 1 + OFFSET) << SHIFT, jnp.float32)
    hw = a_hi - a_lo
    frac = jnp.maximum(sumx - a_lo * cnt, 0.0)
    num = hw * n_gt + frac
    den = ptot + m_gt + 0.5 * neg
    contrib = jnp.where(den > 0.0, num / den, 0.0)
    o_ref[...] = jnp.sum(contrib, keepdims=True) / SAMPLES


def kernel(logits, targets):
    hist_out, p_out = _get_sc_hist()(logits, targets)
    h4 = hist_out.reshape(SAMPLES, 2, 3, NBINS)
    p3 = p_out.reshape(SAMPLES, 2, 16)
    out = pl.pallas_call(
        _tc_body,
        out_shape=jax.ShapeDtypeStruct((1, 1), jnp.float32),
    )(h4, p3)
    return out[0, 0]


# speedup vs baseline: 65.2854x; 1.1085x over previous
import functools

import jax
import jax.numpy as jnp
from jax import lax
from jax.experimental import pallas as pl
from jax.experimental.pallas import tpu as pltpu
from jax.experimental.pallas import tpu_sc as plsc

SAMPLES = 16
ELEMS = 512 * 512
HALF = ELEMS // 2
CHUNK = 16384
NCHUNK = HALF // CHUNK
ITERS = CHUNK // 16
MBITS = 5
NEXP = 16
NBINS = NEXP << MBITS
SHIFT = 23 - MBITS
OFFSET = 119 << MBITS
PLANE = NBINS * 16
HSIZE = 2 * PLANE
HSMALL = 2 * NBINS
UNROLL = 8


def _sc_body(logf, tgtf, hist_out, p_out, buf_l0, buf_t0, buf_l1, buf_t1,
             hcnt, hneg, hsmall, pvec, sl0, st0, sl1, st1):
    s = lax.axis_index("s")
    c = lax.axis_index("c")
    wid = s * 2 + c
    lane = lax.iota(jnp.int32, 16)
    zero16 = jnp.zeros((16,), jnp.float32)
    ones16 = jnp.full((16,), 1.0, jnp.float32)
    lane_adj = (lane - OFFSET * 16, lane - OFFSET * 16 + PLANE)

    def z_body(i, carry):
        for u in range(8):
            hcnt[pl.ds(i * 128 + u * 16, 16)] = zero16
            hneg[pl.ds(i * 128 + u * 16, 16)] = zero16
        return carry

    lax.fori_loop(0, HSIZE // 128, z_body, jnp.int32(0))

    bufs = ((buf_l0, buf_t0, sl0, st0), (buf_l1, buf_t1, sl1, st1))
    ROWS = CHUNK // 512

    def start(ci, parity):
        bl, bt, sl, st = bufs[parity]
        row = c * (HALF // 512) + ci * ROWS
        hl = pltpu.async_copy(logf.at[s, pl.ds(row, ROWS), :], bl, sl)
        ht = pltpu.async_copy(tgtf.at[s, pl.ds(row, ROWS), :], bt, st)
        return hl, ht

    handles = [None, None]
    handles[0] = start(0, 0)
    pacc = zero16
    for ci in range(NCHUNK):
        par = ci % 2
        if ci + 1 < NCHUNK:
            handles[1 - par] = start(ci + 1, 1 - par)
        hl, ht = handles[par]
        hl.wait()
        ht.wait()
        bl, bt = bufs[par][0], bufs[par][1]

        def inner(j, pc, bl=bl, bt=bt):
            psum = zero16
            vals = []
            r = j >> 2
            cg = (j & 3) * 128
            for u in range(UNROLL):
                o = cg + u * 16
                lg = bl[r, pl.ds(o, 16)]
                tg = bt[r, pl.ds(o, 16)]
                m_t = tg != 0
                e = 1.0 - jnp.where(m_t, lg, 0.0 - lg)
                x = jnp.maximum(e, 0.0)
                b = lax.bitcast_convert_type(x, jnp.int32) >> SHIFT
                b = jnp.minimum(jnp.maximum(b, OFFSET), OFFSET + NBINS - 1)
                addr = b * 16 + lane_adj[u % 2]
                m_pos = x > 0.0
                m_neg = jnp.logical_and(m_pos, jnp.logical_not(m_t))
                vals.append((addr, m_pos, m_neg))
                psum = psum + jnp.where(m_t, 1.0, 0.0)
            for addr, m_pos, m_neg in vals:
                plsc.addupdate_scatter(hcnt, [addr], ones16, mask=m_pos)
                plsc.addupdate_scatter(hneg, [addr], ones16, mask=m_neg)
            return pc + psum

        pacc = lax.fori_loop(0, ITERS // UNROLL, inner, pacc)

    pvec[...] = pacc

    def red_body(g, carry):
        b16 = (g * 16 + lane) * 16
        acc_c = zero16
        acc_n = zero16
        for cp in (0, PLANE):
            for l in range(16):
                acc_c = acc_c + plsc.load_gather(hcnt, [b16 + cp + l])
                acc_n = acc_n + plsc.load_gather(hneg, [b16 + cp + l])
        hsmall[pl.ds(g * 16, 16)] = acc_c
        hsmall[pl.ds(NBINS + g * 16, 16)] = acc_n
        return carry

    lax.fori_loop(0, NBINS // 16, red_body, jnp.int32(0))

    pltpu.sync_copy(hsmall, hist_out.at[pl.ds(wid * HSMALL, HSMALL)])
    pltpu.sync_copy(pvec, p_out.at[pl.ds(wid * 16, 16)])


@functools.cache
def _get_sc_hist():
    return functools.partial(
        pl.kernel,
        mesh=plsc.VectorSubcoreMesh(core_axis_name="c", subcore_axis_name="s"),
        compiler_params=pltpu.CompilerParams(needs_layout_passes=False),
        out_type=(
            jax.ShapeDtypeStruct((32 * HSMALL,), jnp.float32),
            jax.ShapeDtypeStruct((32 * 16,), jnp.float32),
        ),
        scratch_types=[
            pltpu.VMEM((CHUNK // 512, 512), jnp.float32),
            pltpu.VMEM((CHUNK // 512, 512), jnp.int32),
            pltpu.VMEM((CHUNK // 512, 512), jnp.float32),
            pltpu.VMEM((CHUNK // 512, 512), jnp.int32),
            pltpu.VMEM((HSIZE,), jnp.float32),
            pltpu.VMEM((HSIZE,), jnp.float32),
            pltpu.VMEM((HSMALL,), jnp.float32),
            pltpu.VMEM((16,), jnp.float32),
            pltpu.SemaphoreType.DMA,
            pltpu.SemaphoreType.DMA,
            pltpu.SemaphoreType.DMA,
            pltpu.SemaphoreType.DMA,
        ],
    )(_sc_body)


def _tc_body(h_ref, p_ref, o_ref):
    h = h_ref[:]
    h2 = h[:, 0] + h[:, 1]
    cnt = h2[:, 0]
    neg = h2[:, 1]
    p = p_ref[:]
    ptot = jnp.sum(jnp.sum(p, axis=2), axis=1, keepdims=True)

    def cum(a):
        sft = 1
        while sft < NBINS:
            a = a + jnp.concatenate(
                [jnp.zeros((SAMPLES, sft), jnp.float32), a[:, :-sft]], axis=1)
            sft *= 2
        return a

    ic = cum(cnt)
    inm = cum(neg)
    n_gt = ic[:, NBINS - 1:] - ic
    m_gt = inm[:, NBINS - 1:] - inm
    k = lax.broadcasted_iota(jnp.int32, (SAMPLES, NBINS), 1)
    a_lo = lax.bitcast_convert_type((k + OFFSET) << SHIFT, jnp.float32)
    a_lo = jnp.where(k == 0, 0.0, a_lo)
    a_hi = lax.bitcast_convert_type((k + 1 + OFFSET) << SHIFT, jnp.float32)
    hw = a_hi - a_lo
    num = hw * (n_gt + 0.5 * cnt)
    den = ptot + m_gt + 0.5 * neg
    contrib = jnp.where(den > 0.0, num / den, 0.0)
    o_ref[...] = jnp.sum(contrib, keepdims=True) / SAMPLES


def kernel(logits, targets):
    hist_out, p_out = _get_sc_hist()(logits, targets)
    h4 = hist_out.reshape(SAMPLES, 2, 2, NBINS)
    p3 = p_out.reshape(SAMPLES, 2, 16)
    out = pl.pallas_call(
        _tc_body,
        out_shape=jax.ShapeDtypeStruct((1, 1), jnp.float32),
    )(h4, p3)
    return out[0, 0]


# speedup vs baseline: 69.4199x; 1.0633x over previous
import functools

import jax
import jax.numpy as jnp
from jax import lax
from jax.experimental import pallas as pl
from jax.experimental.pallas import tpu as pltpu
from jax.experimental.pallas import tpu_sc as plsc

SAMPLES = 16
ELEMS = 512 * 512
HALF = ELEMS // 2
CHUNK = 16384
NCHUNK = HALF // CHUNK
ITERS = CHUNK // 16
MBITS = 5
NEXP = 16
NBINS = NEXP << MBITS
SHIFT = 23 - MBITS
OFFSET = 119 << MBITS
SLOTS = NBINS + 8
PLANE = SLOTS * 16
HSIZE = 2 * PLANE
HSMALL = 2 * NBINS
UNROLL = 8


def _sc_body(logf, tgtf, hist_out, p_out, buf_l0, buf_t0, buf_l1, buf_t1,
             hcnt, hneg, hsmall, pvec, sl0, st0, sl1, st1):
    s = lax.axis_index("s")
    c = lax.axis_index("c")
    wid = s * 2 + c
    lane = lax.iota(jnp.int32, 16)
    zero16 = jnp.zeros((16,), jnp.float32)
    ones16 = jnp.full((16,), 1.0, jnp.float32)
    lane_adj = (lane - (OFFSET - 1) * 16, lane - (OFFSET - 1) * 16 + PLANE)

    def z_body(i, carry):
        for u in range(8):
            hcnt[pl.ds(i * 128 + u * 16, 16)] = zero16
            hneg[pl.ds(i * 128 + u * 16, 16)] = zero16
        return carry

    lax.fori_loop(0, HSIZE // 128, z_body, jnp.int32(0))

    bufs = ((buf_l0, buf_t0, sl0, st0), (buf_l1, buf_t1, sl1, st1))
    ROWS = CHUNK // 512

    def start(ci, parity):
        bl, bt, sl, st = bufs[parity]
        row = c * (HALF // 512) + ci * ROWS
        hl = pltpu.async_copy(logf.at[s, pl.ds(row, ROWS), :], bl, sl)
        ht = pltpu.async_copy(tgtf.at[s, pl.ds(row, ROWS), :], bt, st)
        return hl, ht

    handles = [None, None]
    handles[0] = start(0, 0)
    pacc = jnp.zeros((16,), jnp.int32)
    for ci in range(NCHUNK):
        par = ci % 2
        if ci + 1 < NCHUNK:
            handles[1 - par] = start(ci + 1, 1 - par)
        hl, ht = handles[par]
        hl.wait()
        ht.wait()
        bl, bt = bufs[par][0], bufs[par][1]

        def inner(j, pc, bl=bl, bt=bt):
            psum = pc
            vals = []
            r = j >> 2
            cg = (j & 3) * 128
            for u in range(UNROLL):
                o = cg + u * 16
                lg = bl[r, pl.ds(o, 16)]
                tg = bt[r, pl.ds(o, 16)]
                m_t = tg != 0
                e = 1.0 - jnp.where(m_t, lg, 0.0 - lg)
                b = lax.bitcast_convert_type(e, jnp.int32) >> SHIFT
                b = jnp.minimum(jnp.maximum(b, OFFSET - 1),
                                OFFSET + NBINS - 1)
                addr = b * 16 + lane_adj[u % 2]
                vals.append((addr, jnp.logical_not(m_t)))
                psum = psum + plsc.all_reduce_population_count(m_t)
            for addr, m_nt in vals:
                plsc.addupdate_scatter(hcnt, [addr], ones16)
                plsc.addupdate_scatter(hneg, [addr], ones16, mask=m_nt)
            return psum

        pacc = lax.fori_loop(0, ITERS // UNROLL, inner, pacc)

    pvec[...] = pacc.astype(jnp.float32) * (1.0 / 16.0)

    def red_body(g, carry):
        b16 = (g * 16 + lane + 1) * 16
        acc_c = zero16
        acc_n = zero16
        for cp in (0, PLANE):
            for l in range(16):
                acc_c = acc_c + plsc.load_gather(hcnt, [b16 + cp + l])
                acc_n = acc_n + plsc.load_gather(hneg, [b16 + cp + l])
        hsmall[pl.ds(g * 16, 16)] = acc_c
        hsmall[pl.ds(NBINS + g * 16, 16)] = acc_n
        return carry

    lax.fori_loop(0, NBINS // 16, red_body, jnp.int32(0))

    pltpu.sync_copy(hsmall, hist_out.at[pl.ds(wid * HSMALL, HSMALL)])
    pltpu.sync_copy(pvec, p_out.at[pl.ds(wid * 16, 16)])


@functools.cache
def _get_sc_hist():
    return functools.partial(
        pl.kernel,
        mesh=plsc.VectorSubcoreMesh(core_axis_name="c", subcore_axis_name="s"),
        compiler_params=pltpu.CompilerParams(needs_layout_passes=False),
        out_type=(
            jax.ShapeDtypeStruct((32 * HSMALL,), jnp.float32),
            jax.ShapeDtypeStruct((32 * 16,), jnp.float32),
        ),
        scratch_types=[
            pltpu.VMEM((CHUNK // 512, 512), jnp.float32),
            pltpu.VMEM((CHUNK // 512, 512), jnp.int32),
            pltpu.VMEM((CHUNK // 512, 512), jnp.float32),
            pltpu.VMEM((CHUNK // 512, 512), jnp.int32),
            pltpu.VMEM((HSIZE,), jnp.float32),
            pltpu.VMEM((HSIZE,), jnp.float32),
            pltpu.VMEM((HSMALL,), jnp.float32),
            pltpu.VMEM((16,), jnp.float32),
            pltpu.SemaphoreType.DMA,
            pltpu.SemaphoreType.DMA,
            pltpu.SemaphoreType.DMA,
            pltpu.SemaphoreType.DMA,
        ],
    )(_sc_body)


def _tc_body(h_ref, p_ref, o_ref):
    h = h_ref[:]
    h2 = h[:, 0] + h[:, 1]
    cnt = h2[:, 0]
    neg = h2[:, 1]
    p = p_ref[:]
    ptot = jnp.sum(jnp.sum(p, axis=2), axis=1, keepdims=True)

    def cum(a):
        sft = 1
        while sft < NBINS:
            a = a + jnp.concatenate(
                [jnp.zeros((SAMPLES, sft), jnp.float32), a[:, :-sft]], axis=1)
            sft *= 2
        return a

    ic = cum(cnt)
    inm = cum(neg)
    n_gt = ic[:, NBINS - 1:] - ic
    m_gt = inm[:, NBINS - 1:] - inm
    k = lax.broadcasted_iota(jnp.int32, (SAMPLES, NBINS), 1)
    a_lo = lax.bitcast_convert_type((k + OFFSET) << SHIFT, jnp.float32)
    a_lo = jnp.where(k == 0, 0.0, a_lo)
    a_hi = lax.bitcast_convert_type((k + 1 + OFFSET) << SHIFT, jnp.float32)
    hw = a_hi - a_lo
    num = hw * (n_gt + 0.5 * cnt)
    den = ptot + m_gt + 0.5 * neg
    contrib = jnp.where(den > 0.0, num / den, 0.0)
    o_ref[...] = jnp.sum(contrib, keepdims=True) / SAMPLES


def kernel(logits, targets):
    hist_out, p_out = _get_sc_hist()(logits, targets)
    h4 = hist_out.reshape(SAMPLES, 2, 2, NBINS)
    p3 = p_out.reshape(SAMPLES, 2, 16)
    out = pl.pallas_call(
        _tc_body,
        out_shape=jax.ShapeDtypeStruct((1, 1), jnp.float32),
    )(h4, p3)
    return out[0, 0]


# speedup vs baseline: 75.7666x; 1.0914x over previous
import functools

import jax
import jax.numpy as jnp
from jax import lax
from jax.experimental import pallas as pl
from jax.experimental.pallas import tpu as pltpu
from jax.experimental.pallas import tpu_sc as plsc

SAMPLES = 16
ELEMS = 512 * 512
HALF = ELEMS // 2
CHUNK = 16384
NCHUNK = HALF // CHUNK
ITERS = CHUNK // 16
MBITS = 5
NEXP = 16
NBINS = NEXP << MBITS
SHIFT = 23 - MBITS
OFFSET = 119 << MBITS
SLOTS = NBINS + 16
PLANE = SLOTS * 16
HSIZE = 2 * PLANE
HSMALL = 2 * SLOTS
UNROLL = 8


def _sc_body(logf, tgtf, hist_out, buf_l0, buf_t0, buf_l1, buf_t1,
             hcnt, hneg, hsmall, sl0, st0, sl1, st1):
    s = lax.axis_index("s")
    c = lax.axis_index("c")
    wid = s * 2 + c
    lane = lax.iota(jnp.int32, 16)
    zero16 = jnp.zeros((16,), jnp.float32)
    ones16 = jnp.full((16,), 1.0, jnp.float32)
    lane_adj = (
        lax.bitcast_convert_type(lane - OFFSET * 16, jnp.uint32),
        lax.bitcast_convert_type(lane - OFFSET * 16 + PLANE, jnp.uint32),
    )

    def z_body(i, carry):
        for u in range(8):
            hcnt[pl.ds(i * 128 + u * 16, 16)] = zero16
            hneg[pl.ds(i * 128 + u * 16, 16)] = zero16
        return carry

    lax.fori_loop(0, HSIZE // 128, z_body, jnp.int32(0))

    bufs = ((buf_l0, buf_t0, sl0, st0), (buf_l1, buf_t1, sl1, st1))
    ROWS = CHUNK // 512

    def start(ci, parity):
        bl, bt, sl, st = bufs[parity]
        row = c * (HALF // 512) + ci * ROWS
        hl = pltpu.async_copy(logf.at[s, pl.ds(row, ROWS), :], bl, sl)
        ht = pltpu.async_copy(tgtf.at[s, pl.ds(row, ROWS), :], bt, st)
        return hl, ht

    handles = [None, None]
    handles[0] = start(0, 0)
    for ci in range(NCHUNK):
        par = ci % 2
        if ci + 1 < NCHUNK:
            handles[1 - par] = start(ci + 1, 1 - par)
        hl, ht = handles[par]
        hl.wait()
        ht.wait()
        bl, bt = bufs[par][0], bufs[par][1]

        def inner(j, pc, bl=bl, bt=bt):
            vals = []
            r = j >> 2
            cg = (j & 3) * 128
            for u in range(UNROLL):
                o = cg + u * 16
                lg = bl[r, pl.ds(o, 16)]
                tg = bt[r, pl.ds(o, 16)]
                m_t = tg != 0
                e = 1.0 - jnp.where(m_t, lg, 0.0 - lg)
                bu = lax.bitcast_convert_type(e, jnp.uint32) >> SHIFT
                bu = jnp.minimum(jnp.maximum(bu, jnp.uint32(OFFSET)),
                                 jnp.uint32(OFFSET + NBINS))
                addr = lax.bitcast_convert_type(
                    bu * 16 + lane_adj[u % 2], jnp.int32)
                vals.append((addr, jnp.logical_not(m_t)))
            for addr, m_nt in vals:
                plsc.addupdate_scatter(hcnt, [addr], ones16)
                plsc.addupdate_scatter(hneg, [addr], ones16, mask=m_nt)
            return pc

        lax.fori_loop(0, ITERS // UNROLL, inner, jnp.int32(0))

    def red_body(g, carry):
        b16 = (g * 16 + lane) * 16
        acc_c = zero16
        acc_n = zero16
        for cp in (0, PLANE):
            for l in range(16):
                acc_c = acc_c + plsc.load_gather(hcnt, [b16 + cp + l])
                acc_n = acc_n + plsc.load_gather(hneg, [b16 + cp + l])
        hsmall[pl.ds(g * 16, 16)] = acc_c
        hsmall[pl.ds(SLOTS + g * 16, 16)] = acc_n
        return carry

    lax.fori_loop(0, SLOTS // 16, red_body, jnp.int32(0))

    pltpu.sync_copy(hsmall, hist_out.at[pl.ds(wid * HSMALL, HSMALL)])


@functools.cache
def _get_sc_hist():
    return functools.partial(
        pl.kernel,
        mesh=plsc.VectorSubcoreMesh(core_axis_name="c", subcore_axis_name="s"),
        compiler_params=pltpu.CompilerParams(needs_layout_passes=False),
        out_type=jax.ShapeDtypeStruct((32 * HSMALL,), jnp.float32),
        scratch_types=[
            pltpu.VMEM((CHUNK // 512, 512), jnp.float32),
            pltpu.VMEM((CHUNK // 512, 512), jnp.int32),
            pltpu.VMEM((CHUNK // 512, 512), jnp.float32),
            pltpu.VMEM((CHUNK // 512, 512), jnp.int32),
            pltpu.VMEM((HSIZE,), jnp.float32),
            pltpu.VMEM((HSIZE,), jnp.float32),
            pltpu.VMEM((HSMALL,), jnp.float32),
            pltpu.SemaphoreType.DMA,
            pltpu.SemaphoreType.DMA,
            pltpu.SemaphoreType.DMA,
            pltpu.SemaphoreType.DMA,
        ],
    )(_sc_body)


def _tc_body(h_ref, o_ref):
    h = h_ref[:]
    h2 = h[:, 0] + h[:, 1]
    cnt = h2[:, 0, :NBINS]
    neg = h2[:, 1, :NBINS]
    ptot = ELEMS - jnp.sum(h2[:, 1], axis=1, keepdims=True)

    def cum(a):
        sft = 1
        while sft < NBINS:
            a = a + jnp.concatenate(
                [jnp.zeros((SAMPLES, sft), jnp.float32), a[:, :-sft]], axis=1)
            sft *= 2
        return a

    ic = cum(cnt)
    inm = cum(neg)
    n_gt = ic[:, NBINS - 1:] - ic
    m_gt = inm[:, NBINS - 1:] - inm
    k = lax.broadcasted_iota(jnp.int32, (SAMPLES, NBINS), 1)
    a_lo = lax.bitcast_convert_type((k + OFFSET) << SHIFT, jnp.float32)
    a_lo = jnp.where(k == 0, 0.0, a_lo)
    a_hi = lax.bitcast_convert_type((k + 1 + OFFSET) << SHIFT, jnp.float32)
    hw = a_hi - a_lo
    num = hw * (n_gt + 0.5 * cnt)
    den = ptot + m_gt + 0.5 * neg
    contrib = jnp.where(den > 0.0, num / den, 0.0)
    o_ref[...] = jnp.sum(contrib, keepdims=True) / SAMPLES


def kernel(logits, targets):
    hist_out = _get_sc_hist()(logits, targets)
    h4 = hist_out.reshape(SAMPLES, 2, 2, SLOTS)
    out = pl.pallas_call(
        _tc_body,
        out_shape=jax.ShapeDtypeStruct((1, 1), jnp.float32),
    )(h4)
    return out[0, 0]


# speedup vs baseline: 78.7490x; 1.0394x over previous
import functools

import jax
import jax.numpy as jnp
from jax import lax
from jax.experimental import pallas as pl
from jax.experimental.pallas import tpu as pltpu
from jax.experimental.pallas import tpu_sc as plsc

SAMPLES = 16
ELEMS = 512 * 512
HALF = ELEMS // 2
CHUNK = 16384
NCHUNK = HALF // CHUNK
ITERS = CHUNK // 16
MBITS = 5
NEXP = 16
NBINS = NEXP << MBITS
SHIFT = 23 - MBITS
OFFSET = 119 << MBITS
SLOTS = NBINS + 16
PLANE = SLOTS * 16
HSIZE = 2 * PLANE
HSMALL = 2 * SLOTS
UNROLL = 8


def _sc_body(logf, tgtf, hist_out, buf_l0, buf_t0, buf_l1, buf_t1,
             hcnt, hneg, hsmall, sl0, st0, sl1, st1):
    s = lax.axis_index("s")
    c = lax.axis_index("c")
    wid = s * 2 + c
    lane = lax.iota(jnp.int32, 16)
    zero16 = jnp.zeros((16,), jnp.float32)
    ones16 = jnp.full((16,), 1.0, jnp.float32)
    lane_adj = (
        lax.bitcast_convert_type(lane - OFFSET * 16, jnp.uint32),
        lax.bitcast_convert_type(lane - OFFSET * 16 + PLANE, jnp.uint32),
    )

    bufs = ((buf_l0, buf_t0, sl0, st0), (buf_l1, buf_t1, sl1, st1))
    ROWS = CHUNK // 512

    def start(ci, parity):
        bl, bt, sl, st = bufs[parity]
        row = c * (HALF // 512) + ci * ROWS
        hl = pltpu.async_copy(logf.at[s, pl.ds(row, ROWS), :], bl, sl)
        ht = pltpu.async_copy(tgtf.at[s, pl.ds(row, ROWS), :], bt, st)
        return hl, ht

    handles = [None, None]
    handles[0] = start(0, 0)

    def z_body(i, carry):
        for u in range(8):
            hcnt[pl.ds(i * 128 + u * 16, 16)] = zero16
            hneg[pl.ds(i * 128 + u * 16, 16)] = zero16
        return carry

    lax.fori_loop(0, HSIZE // 128, z_body, jnp.int32(0))
    for ci in range(NCHUNK):
        par = ci % 2
        if ci + 1 < NCHUNK:
            handles[1 - par] = start(ci + 1, 1 - par)
        hl, ht = handles[par]
        hl.wait()
        ht.wait()
        bl, bt = bufs[par][0], bufs[par][1]

        def inner(j, pc, bl=bl, bt=bt):
            vals = []
            r = j >> 2
            cg = (j & 3) * 128
            for u in range(UNROLL):
                o = cg + u * 16
                lg = bl[r, pl.ds(o, 16)]
                tg = bt[r, pl.ds(o, 16)]
                m_t = tg != 0
                e = 1.0 - jnp.where(m_t, lg, 0.0 - lg)
                bu = lax.bitcast_convert_type(e, jnp.uint32) >> SHIFT
                bu = jnp.minimum(jnp.maximum(bu, jnp.uint32(OFFSET)),
                                 jnp.uint32(OFFSET + NBINS))
                addr = lax.bitcast_convert_type(
                    bu * 16 + lane_adj[u % 2], jnp.int32)
                vals.append((addr, jnp.logical_not(m_t)))
            for addr, m_nt in vals:
                plsc.addupdate_scatter(hcnt, [addr], ones16)
                plsc.addupdate_scatter(hneg, [addr], ones16, mask=m_nt)
            return pc

        lax.fori_loop(0, ITERS // UNROLL, inner, jnp.int32(0))

    def red_body(g, carry):
        b16 = (g * 16 + lane) * 16
        acc_c = zero16
        acc_n = zero16
        for cp in (0, PLANE):
            for l in range(16):
                acc_c = acc_c + plsc.load_gather(hcnt, [b16 + cp + l])
                acc_n = acc_n + plsc.load_gather(hneg, [b16 + cp + l])
        hsmall[pl.ds(g * 16, 16)] = acc_c
        hsmall[pl.ds(SLOTS + g * 16, 16)] = acc_n
        return carry

    lax.fori_loop(0, SLOTS // 16, red_body, jnp.int32(0))

    pltpu.sync_copy(hsmall, hist_out.at[pl.ds(wid * HSMALL, HSMALL)])


@functools.cache
def _get_sc_hist():
    return functools.partial(
        pl.kernel,
        mesh=plsc.VectorSubcoreMesh(core_axis_name="c", subcore_axis_name="s"),
        compiler_params=pltpu.CompilerParams(needs_layout_passes=False),
        out_type=jax.ShapeDtypeStruct((32 * HSMALL,), jnp.float32),
        scratch_types=[
            pltpu.VMEM((CHUNK // 512, 512), jnp.float32),
            pltpu.VMEM((CHUNK // 512, 512), jnp.int32),
            pltpu.VMEM((CHUNK // 512, 512), jnp.float32),
            pltpu.VMEM((CHUNK // 512, 512), jnp.int32),
            pltpu.VMEM((HSIZE,), jnp.float32),
            pltpu.VMEM((HSIZE,), jnp.float32),
            pltpu.VMEM((HSMALL,), jnp.float32),
            pltpu.SemaphoreType.DMA,
            pltpu.SemaphoreType.DMA,
            pltpu.SemaphoreType.DMA,
            pltpu.SemaphoreType.DMA,
        ],
    )(_sc_body)


def _tc_body(h_ref, o_ref):
    h = h_ref[:]
    h2 = h[:, 0] + h[:, 1]
    cnt = h2[:, 0, :NBINS]
    neg = h2[:, 1, :NBINS]
    ptot = ELEMS - jnp.sum(h2[:, 1], axis=1, keepdims=True)

    def cum(a):
        sft = 1
        while sft < NBINS:
            a = a + jnp.concatenate(
                [jnp.zeros((SAMPLES, sft), jnp.float32), a[:, :-sft]], axis=1)
            sft *= 2
        return a

    ic = cum(cnt)
    inm = cum(neg)
    n_gt = ic[:, NBINS - 1:] - ic
    m_gt = inm[:, NBINS - 1:] - inm
    k = lax.broadcasted_iota(jnp.int32, (SAMPLES, NBINS), 1)
    a_lo = lax.bitcast_convert_type((k + OFFSET) << SHIFT, jnp.float32)
    a_lo = jnp.where(k == 0, 0.0, a_lo)
    a_hi = lax.bitcast_convert_type((k + 1 + OFFSET) << SHIFT, jnp.float32)
    hw = a_hi - a_lo
    num = hw * (n_gt + 0.5 * cnt)
    den = ptot + m_gt + 0.5 * neg
    contrib = jnp.where(den > 0.0, num / den, 0.0)
    o_ref[...] = jnp.sum(contrib, keepdims=True) / SAMPLES


def kernel(logits, targets):
    hist_out = _get_sc_hist()(logits, targets)
    h4 = hist_out.reshape(SAMPLES, 2, 2, SLOTS)
    out = pl.pallas_call(
        _tc_body,
        out_shape=jax.ShapeDtypeStruct((1, 1), jnp.float32),
    )(h4)
    return out[0, 0]


# speedup vs baseline: 83.9693x; 1.0663x over previous
import functools

import jax
import jax.numpy as jnp
from jax import lax
from jax.experimental import pallas as pl
from jax.experimental.pallas import tpu as pltpu
from jax.experimental.pallas import tpu_sc as plsc

SAMPLES = 16
ELEMS = 512 * 512
HALF = ELEMS // 2
CHUNK = 16384
NCHUNK = HALF // CHUNK
ITERS = CHUNK // 16
MBITS = 5
NEXP = 16
NBINS = NEXP << MBITS
SHIFT = 23 - MBITS
OFFSET = 119 << MBITS
SLOTS = NBINS + 16
PLANE = SLOTS * 16
HSIZE = 2 * PLANE
HSMALL = 2 * SLOTS
UNROLL = 16


def _sc_body(logf, tgtf, hist_out, buf_l0, buf_t0, buf_l1, buf_t1,
             hcnt, hneg, hsmall, sl0, st0, sl1, st1):
    s = lax.axis_index("s")
    c = lax.axis_index("c")
    wid = s * 2 + c
    lane = lax.iota(jnp.int32, 16)
    zero16 = jnp.zeros((16,), jnp.float32)
    ones16 = jnp.full((16,), 1.0, jnp.float32)
    lane_adj = (
        lax.bitcast_convert_type(lane - OFFSET * 16, jnp.uint32),
        lax.bitcast_convert_type(lane - OFFSET * 16 + PLANE, jnp.uint32),
    )

    bufs = ((buf_l0, buf_t0, sl0, st0), (buf_l1, buf_t1, sl1, st1))
    ROWS = CHUNK // 512

    def start(ci, parity):
        bl, bt, sl, st = bufs[parity]
        row = c * (HALF // 512) + ci * ROWS
        hl = pltpu.async_copy(logf.at[s, pl.ds(row, ROWS), :], bl, sl)
        ht = pltpu.async_copy(tgtf.at[s, pl.ds(row, ROWS), :], bt, st)
        return hl, ht

    handles = [None, None]
    handles[0] = start(0, 0)

    def z_body(i, carry):
        for u in range(8):
            hcnt[pl.ds(i * 128 + u * 16, 16)] = zero16
            hneg[pl.ds(i * 128 + u * 16, 16)] = zero16
        return carry

    lax.fori_loop(0, HSIZE // 128, z_body, jnp.int32(0))
    for ci in range(NCHUNK):
        par = ci % 2
        if ci + 1 < NCHUNK:
            handles[1 - par] = start(ci + 1, 1 - par)
        hl, ht = handles[par]
        hl.wait()
        ht.wait()
        bl, bt = bufs[par][0], bufs[par][1]

        def inner(j, pc, bl=bl, bt=bt):
            vals = []
            gpr_log2 = {1: 0, 2: 1, 4: 2, 8: 3, 16: 4, 32: 5}[32 // UNROLL]
            r = j >> gpr_log2
            cg = (j & ((32 // UNROLL) - 1)) * (UNROLL * 16)
            for u in range(UNROLL):
                o = cg + u * 16
                lg = bl[r, pl.ds(o, 16)]
                tg = bt[r, pl.ds(o, 16)]
                m_t = tg != 0
                e = 1.0 - jnp.where(m_t, lg, 0.0 - lg)
                bu = lax.bitcast_convert_type(e, jnp.uint32) >> SHIFT
                bu = jnp.minimum(jnp.maximum(bu, jnp.uint32(OFFSET)),
                                 jnp.uint32(OFFSET + NBINS))
                addr = lax.bitcast_convert_type(
                    bu * 16 + lane_adj[u % 2], jnp.int32)
                vals.append((addr, jnp.logical_not(m_t)))
            for addr, m_nt in vals:
                plsc.addupdate_scatter(hcnt, [addr], ones16)
                plsc.addupdate_scatter(hneg, [addr], ones16, mask=m_nt)
            return pc

        lax.fori_loop(0, ITERS // UNROLL, inner, jnp.int32(0))

    def red_body(g, carry):
        b16 = (g * 16 + lane) * 16
        acc_c = zero16
        acc_n = zero16
        for cp in (0, PLANE):
            for l in range(16):
                acc_c = acc_c + plsc.load_gather(hcnt, [b16 + cp + l])
                acc_n = acc_n + plsc.load_gather(hneg, [b16 + cp + l])
        hsmall[pl.ds(g * 16, 16)] = acc_c
        hsmall[pl.ds(SLOTS + g * 16, 16)] = acc_n
        return carry

    lax.fori_loop(0, SLOTS // 16, red_body, jnp.int32(0))

    pltpu.sync_copy(hsmall, hist_out.at[pl.ds(wid * HSMALL, HSMALL)])


@functools.cache
def _get_sc_hist():
    return functools.partial(
        pl.kernel,
        mesh=plsc.VectorSubcoreMesh(core_axis_name="c", subcore_axis_name="s"),
        compiler_params=pltpu.CompilerParams(needs_layout_passes=False),
        out_type=jax.ShapeDtypeStruct((32 * HSMALL,), jnp.float32),
        scratch_types=[
            pltpu.VMEM((CHUNK // 512, 512), jnp.float32),
            pltpu.VMEM((CHUNK // 512, 512), jnp.int32),
            pltpu.VMEM((CHUNK // 512, 512), jnp.float32),
            pltpu.VMEM((CHUNK // 512, 512), jnp.int32),
            pltpu.VMEM((HSIZE,), jnp.float32),
            pltpu.VMEM((HSIZE,), jnp.float32),
            pltpu.VMEM((HSMALL,), jnp.float32),
            pltpu.SemaphoreType.DMA,
            pltpu.SemaphoreType.DMA,
            pltpu.SemaphoreType.DMA,
            pltpu.SemaphoreType.DMA,
        ],
    )(_sc_body)


def _tc_body(h_ref, o_ref):
    h = h_ref[:]
    h2 = h[:, 0] + h[:, 1]
    cnt = h2[:, 0, :NBINS]
    neg = h2[:, 1, :NBINS]
    ptot = ELEMS - jnp.sum(h2[:, 1], axis=1, keepdims=True)

    def cum(a):
        sft = 1
        while sft < NBINS:
            a = a + jnp.concatenate(
                [jnp.zeros((SAMPLES, sft), jnp.float32), a[:, :-sft]], axis=1)
            sft *= 2
        return a

    ic = cum(cnt)
    inm = cum(neg)
    n_gt = ic[:, NBINS - 1:] - ic
    m_gt = inm[:, NBINS - 1:] - inm
    k = lax.broadcasted_iota(jnp.int32, (SAMPLES, NBINS), 1)
    a_lo = lax.bitcast_convert_type((k + OFFSET) << SHIFT, jnp.float32)
    a_lo = jnp.where(k == 0, 0.0, a_lo)
    a_hi = lax.bitcast_convert_type((k + 1 + OFFSET) << SHIFT, jnp.float32)
    hw = a_hi - a_lo
    num = hw * (n_gt + 0.5 * cnt)
    den = ptot + m_gt + 0.5 * neg
    contrib = jnp.where(den > 0.0, num / den, 0.0)
    o_ref[...] = jnp.sum(contrib, keepdims=True) / SAMPLES


def kernel(logits, targets):
    hist_out = _get_sc_hist()(logits, targets)
    h4 = hist_out.reshape(SAMPLES, 2, 2, SLOTS)
    out = pl.pallas_call(
        _tc_body,
        out_shape=jax.ShapeDtypeStruct((1, 1), jnp.float32),
    )(h4)
    return out[0, 0]
